# Initial kernel scaffold; baseline (speedup 1.0000x reference)
import functools

import jax
import jax.numpy as jnp
from jax import lax
from jax.experimental import pallas as pl
from jax.experimental.pallas import tpu as pltpu
from jax.experimental.pallas import tpu_sc as plsc

N = 10000
E = 160000
D_IN = 256
DIM = 512
L = 4
G = 64

NC = 2
NS = 16
CW = 128
EB = 128
SENT = 8

E_PAD = ((E + NS * EB - 1) // (NS * EB)) * (NS * EB)
BN = 1000
NB = N // BN


def _make_agg(C):
    chunks_per_core = C // NC
    ebs = E_PAD // NS
    nb = ebs // EB
    rps = N // NS

    mesh = plsc.VectorSubcoreMesh(
        core_axis_name="c", subcore_axis_name="s", num_cores=NC, num_subcores=NS
    )

    @functools.partial(
        pl.kernel,
        out_type=jax.ShapeDtypeStruct((C * N, CW), jnp.float32),
        mesh=mesh,
        scratch_types=[
            pltpu.VMEM((EB,), jnp.int32),
            pltpu.VMEM((1, EB), jnp.int32),
            pltpu.VMEM((EB, CW), jnp.float32),
            pltpu.VMEM_SHARED((N + SENT, CW), jnp.float32),
            pltpu.SemaphoreType.DMA,
        ],
    )
    def agg(h_hbm, dstoff_hbm, src_hbm, out_hbm, dst_v, src_v, rows_v, acc_sh, sem):
        cid = lax.axis_index("c")
        sid = lax.axis_index("s")

        def zrow(r, _):
            for j in range(CW // 16):
                rows_v[r, pl.ds(j * 16, 16)] = jnp.zeros((16,), jnp.float32)
            return 0
        lax.fori_loop(0, EB, zrow, 0)

        for k in range(chunks_per_core):
            chunk = cid * chunks_per_core + k

            nfull = rps // EB
            for t in range(nfull):
                pltpu.sync_copy(rows_v, acc_sh.at[pl.ds(sid * rps + t * EB, EB)])
            rem = rps - nfull * EB
            if rem:
                pltpu.sync_copy(
                    rows_v.at[pl.ds(0, rem)],
                    acc_sh.at[pl.ds(sid * rps + nfull * EB, rem)],
                )
            plsc.subcore_barrier()

            def body(b, _):
                off = sid * ebs + b * EB
                pltpu.sync_copy(dstoff_hbm.at[chunk, pl.ds(off, EB)], dst_v)
                pltpu.sync_copy(src_hbm.at[pl.ds(off, EB)], src_v.at[0])
                pltpu.async_copy(h_hbm.at[dst_v], rows_v, sem).wait()
                pltpu.sync_copy(rows_v, acc_sh.at[src_v.at[0]], add=True)
                return 0
            lax.fori_loop(0, nb, body, 0)
            plsc.subcore_barrier()

            pltpu.sync_copy(
                acc_sh.at[pl.ds(sid * rps, rps)],
                out_hbm.at[pl.ds(chunk * N + sid * rps, rps)],
            )
            if k + 1 < chunks_per_core:
                plsc.subcore_barrier()

    return agg


def _make_mlp(C):
    din = C * CW

    def body(scale_ref, pooled_ref, h_ref, w1_ref, b1_ref, w2_ref, b2_ref,
             hh_ref, stats_ref, acc_ref):
        i = pl.program_id(0)
        s = scale_ref[0, 0]
        p = jnp.concatenate([pooled_ref[c] for c in range(C)], axis=-1)
        hcat = jnp.concatenate([h_ref[c] for c in range(C)], axis=-1)
        a = p + s * hcat
        z = jnp.dot(a, w1_ref[...], preferred_element_type=jnp.float32)
        z = jnp.maximum(z + b1_ref[0][None, :], 0.0)
        hh = jnp.dot(z, w2_ref[...], preferred_element_type=jnp.float32)
        hh = jnp.maximum(hh + b2_ref[0][None, :], 0.0)
        hh_ref[...] = hh

        @pl.when(i == 0)
        def _():
            acc_ref[...] = jnp.zeros_like(acc_ref)
        acc_ref[0, :] += jnp.sum(hh, axis=0)
        acc_ref[1, :] += jnp.sum(hh * hh, axis=0)

        @pl.when(i == NB - 1)
        def _():
            stats_ref[...] = acc_ref[...]

    return pl.pallas_call(
        body,
        grid=(NB,),
        in_specs=[
            pl.BlockSpec(memory_space=pltpu.SMEM),
            pl.BlockSpec((C, BN, CW), lambda i: (0, i, 0)),
            pl.BlockSpec((C, BN, CW), lambda i: (0, i, 0)),
            pl.BlockSpec((C * CW, DIM), lambda i: (0, 0)),
            pl.BlockSpec((1, DIM), lambda i: (0, 0)),
            pl.BlockSpec((DIM, DIM), lambda i: (0, 0)),
            pl.BlockSpec((1, DIM), lambda i: (0, 0)),
        ],
        out_specs=[
            pl.BlockSpec((BN, DIM), lambda i: (i, 0)),
            pl.BlockSpec((2, DIM), lambda i: (0, 0)),
        ],
        out_shape=[
            jax.ShapeDtypeStruct((N, DIM), jnp.float32),
            jax.ShapeDtypeStruct((2, DIM), jnp.float32),
        ],
        scratch_shapes=[pltpu.VMEM((2, DIM), jnp.float32)],
    )


def _make_norm(with_relayout):
    C = DIM // CW

    def body(hh_ref, stats_ref, gamma_ref, beta_ref, batch_ref, *refs):
        if with_relayout:
            xs_ref, h4_ref, pool_ref, acc_ref = refs
        else:
            xs_ref, pool_ref, acc_ref = refs
        i = pl.program_id(0)
        mean = stats_ref[0, :] * (1.0 / N)
        var = stats_ref[1, :] * (1.0 / N) - mean * mean
        g = gamma_ref[0, :] * lax.rsqrt(var + 1e-5)
        b = beta_ref[0, :] - mean * g
        hn = hh_ref[...] * g[None, :] + b[None, :]
        xs_ref[...] = hn
        if with_relayout:
            for c in range(C):
                h4_ref[c] = hn[:, c * CW:(c + 1) * CW]
        bid = batch_ref[0, 0, :]
        onehot = (bid[:, None] == lax.broadcasted_iota(jnp.int32, (1, G), 1)
                  ).astype(jnp.float32)

        @pl.when(i == 0)
        def _():
            acc_ref[...] = jnp.zeros_like(acc_ref)
        acc_ref[...] += lax.dot_general(
            onehot, hn, (((0,), (0,)), ((), ())),
            preferred_element_type=jnp.float32)

        @pl.when(i == NB - 1)
        def _():
            pool_ref[...] = acc_ref[...]

    out_specs = [pl.BlockSpec((BN, DIM), lambda i: (i, 0))]
    out_shape = [jax.ShapeDtypeStruct((N, DIM), jnp.float32)]
    if with_relayout:
        out_specs.append(pl.BlockSpec((C, BN, CW), lambda i: (0, i, 0)))
        out_shape.append(jax.ShapeDtypeStruct((C, N, CW), jnp.float32))
    out_specs.append(pl.BlockSpec((G, DIM), lambda i: (0, 0)))
    out_shape.append(jax.ShapeDtypeStruct((G, DIM), jnp.float32))

    return pl.pallas_call(
        body,
        grid=(NB,),
        in_specs=[
            pl.BlockSpec((BN, DIM), lambda i: (i, 0)),
            pl.BlockSpec((2, DIM), lambda i: (0, 0)),
            pl.BlockSpec((1, DIM), lambda i: (0, 0)),
            pl.BlockSpec((1, DIM), lambda i: (0, 0)),
            pl.BlockSpec((1, 1, BN), lambda i: (i, 0, 0)),
        ],
        out_specs=out_specs,
        out_shape=out_shape,
        scratch_shapes=[pltpu.VMEM((G, DIM), jnp.float32)],
    )


@jax.jit
def kernel(x, edge_index, batch, eps, params):
    src = edge_index[0]
    dst = edge_index[1]
    pad = E_PAD - E
    dst_p = jnp.pad(dst, (0, pad))
    src_p = jnp.pad(src, (0, pad), constant_values=N)
    coff = (jnp.arange(4, dtype=jnp.int32) * N)[:, None]
    dstoff4 = dst_p[None, :] + coff
    dstoff2 = dstoff4[:2]
    batch3 = batch.reshape(NB, 1, BN)

    agg2 = _make_agg(2)
    agg4 = _make_agg(4)
    norm_mid = _make_norm(True)
    norm_last = _make_norm(False)

    h4 = x.reshape(N, 2, CW).transpose(1, 0, 2)
    xs, pools = [], []
    for i, p in enumerate(params):
        C = 2 if i == 0 else 4
        agg = agg2 if i == 0 else agg4
        dstoff = dstoff2 if i == 0 else dstoff4
        pooled = agg(h4.reshape(C * N, CW), dstoff, src_p).reshape(C, N, CW)
        scale = (1.0 + eps[i]).reshape(1, 1)
        hh, stats = _make_mlp(C)(
            scale, pooled, h4,
            p["W1"], p["b1"].reshape(1, DIM),
            p["W2"], p["b2"].reshape(1, DIM),
        )
        gam = p["gamma"].reshape(1, DIM)
        bet = p["beta"].reshape(1, DIM)
        if i < L - 1:
            h, h4, pool = norm_mid(hh, stats, gam, bet, batch3)
        else:
            h, pool = norm_last(hh, stats, gam, bet, batch3)
        xs.append(h)
        pools.append(pool)
    return (jnp.concatenate(pools, axis=1), jnp.concatenate(xs, axis=1))


# speedup vs baseline: 2.6820x; 2.6820x over previous
import functools

import jax
import jax.numpy as jnp
from jax import lax
from jax.experimental import pallas as pl
from jax.experimental.pallas import tpu as pltpu
from jax.experimental.pallas import tpu_sc as plsc

N = 10000
E = 160000
D_IN = 256
DIM = 512
L = 4
G = 64

NC = 2
NS = 16
CW = 128
EB = 128
SENT = 8

E_PAD = ((E + NS * EB - 1) // (NS * EB)) * (NS * EB)
BN = 1000
NB = N // BN


def _make_agg(C):
    chunks_per_core = C // NC
    ebs = E_PAD // NS
    nb = ebs // EB
    rps = N // NS

    mesh = plsc.VectorSubcoreMesh(
        core_axis_name="c", subcore_axis_name="s", num_cores=NC, num_subcores=NS
    )

    @functools.partial(
        pl.kernel,
        out_type=jax.ShapeDtypeStruct((C * N, CW), jnp.float32),
        mesh=mesh,
        compiler_params=pltpu.CompilerParams(use_tc_tiling_on_sc=False),
        scratch_types=[
            pltpu.VMEM((EB,), jnp.int32),
            pltpu.VMEM((1, EB), jnp.int32),
            pltpu.VMEM((EB, CW), jnp.float32),
            pltpu.VMEM((EB, CW), jnp.float32),
            pltpu.VMEM_SHARED((N + SENT, CW), jnp.float32),
            pltpu.SemaphoreType.DMA,
        ],
    )
    def agg(h_hbm, dstoff_hbm, src_hbm, out_hbm, dst_v, src_v, rows_v, zero_v, acc_sh, sem):
        cid = lax.axis_index("c")
        sid = lax.axis_index("s")

        def zrow(r, _):
            for j in range(CW // 16):
                zero_v[r, pl.ds(j * 16, 16)] = jnp.zeros((16,), jnp.float32)
            return 0
        lax.fori_loop(0, EB, zrow, 0)

        for k in range(chunks_per_core):
            chunk = cid * chunks_per_core + k

            nfull = rps // EB
            for t in range(nfull):
                pltpu.sync_copy(zero_v, acc_sh.at[pl.ds(sid * rps + t * EB, EB)])
            rem = rps - nfull * EB
            if rem:
                pltpu.sync_copy(
                    zero_v.at[pl.ds(0, rem)],
                    acc_sh.at[pl.ds(sid * rps + nfull * EB, rem)],
                )
            plsc.subcore_barrier()

            def body(b, _):
                off = sid * ebs + b * EB
                pltpu.sync_copy(dstoff_hbm.at[chunk, pl.ds(off, EB)], dst_v)
                pltpu.sync_copy(src_hbm.at[pl.ds(off, EB)], src_v.at[0])
                pltpu.async_copy(h_hbm.at[dst_v], rows_v, sem).wait()
                pltpu.sync_copy(rows_v, acc_sh.at[src_v.at[0]], add=True)
                return 0
            lax.fori_loop(0, nb, body, 0)
            plsc.subcore_barrier()

            pltpu.sync_copy(
                acc_sh.at[pl.ds(sid * rps, rps)],
                out_hbm.at[pl.ds(chunk * N + sid * rps, rps)],
            )
            if k + 1 < chunks_per_core:
                plsc.subcore_barrier()

    return agg


def _make_mlp(C):
    din = C * CW

    def body(scale_ref, pooled_ref, h_ref, w1_ref, b1_ref, w2_ref, b2_ref,
             hh_ref, stats_ref, acc_ref):
        i = pl.program_id(0)
        s = scale_ref[0, 0]
        p = jnp.concatenate([pooled_ref[c] for c in range(C)], axis=-1)
        hcat = jnp.concatenate([h_ref[c] for c in range(C)], axis=-1)
        a = p + s * hcat
        z = jnp.dot(a, w1_ref[...], preferred_element_type=jnp.float32)
        z = jnp.maximum(z + b1_ref[0][None, :], 0.0)
        hh = jnp.dot(z, w2_ref[...], preferred_element_type=jnp.float32)
        hh = jnp.maximum(hh + b2_ref[0][None, :], 0.0)
        hh_ref[...] = hh

        @pl.when(i == 0)
        def _():
            acc_ref[...] = jnp.zeros_like(acc_ref)
        acc_ref[0, :] += jnp.sum(hh, axis=0)
        acc_ref[1, :] += jnp.sum(hh * hh, axis=0)

        @pl.when(i == NB - 1)
        def _():
            stats_ref[...] = acc_ref[...]

    return pl.pallas_call(
        body,
        grid=(NB,),
        in_specs=[
            pl.BlockSpec(memory_space=pltpu.SMEM),
            pl.BlockSpec((C, BN, CW), lambda i: (0, i, 0)),
            pl.BlockSpec((C, BN, CW), lambda i: (0, i, 0)),
            pl.BlockSpec((C * CW, DIM), lambda i: (0, 0)),
            pl.BlockSpec((1, DIM), lambda i: (0, 0)),
            pl.BlockSpec((DIM, DIM), lambda i: (0, 0)),
            pl.BlockSpec((1, DIM), lambda i: (0, 0)),
        ],
        out_specs=[
            pl.BlockSpec((BN, DIM), lambda i: (i, 0)),
            pl.BlockSpec((2, DIM), lambda i: (0, 0)),
        ],
        out_shape=[
            jax.ShapeDtypeStruct((N, DIM), jnp.float32),
            jax.ShapeDtypeStruct((2, DIM), jnp.float32),
        ],
        scratch_shapes=[pltpu.VMEM((2, DIM), jnp.float32)],
    )


def _make_norm(with_relayout):
    C = DIM // CW

    def body(hh_ref, stats_ref, gamma_ref, beta_ref, batch_ref, *refs):
        if with_relayout:
            xs_ref, h4_ref, pool_ref, acc_ref = refs
        else:
            xs_ref, pool_ref, acc_ref = refs
        i = pl.program_id(0)
        mean = stats_ref[0, :] * (1.0 / N)
        var = stats_ref[1, :] * (1.0 / N) - mean * mean
        g = gamma_ref[0, :] * lax.rsqrt(var + 1e-5)
        b = beta_ref[0, :] - mean * g
        hn = hh_ref[...] * g[None, :] + b[None, :]
        xs_ref[...] = hn
        if with_relayout:
            for c in range(C):
                h4_ref[c] = hn[:, c * CW:(c + 1) * CW]
        bid = batch_ref[0, 0, :]
        onehot = (bid[:, None] == lax.broadcasted_iota(jnp.int32, (1, G), 1)
                  ).astype(jnp.float32)

        @pl.when(i == 0)
        def _():
            acc_ref[...] = jnp.zeros_like(acc_ref)
        acc_ref[...] += lax.dot_general(
            onehot, hn, (((0,), (0,)), ((), ())),
            preferred_element_type=jnp.float32)

        @pl.when(i == NB - 1)
        def _():
            pool_ref[...] = acc_ref[...]

    out_specs = [pl.BlockSpec((BN, DIM), lambda i: (i, 0))]
    out_shape = [jax.ShapeDtypeStruct((N, DIM), jnp.float32)]
    if with_relayout:
        out_specs.append(pl.BlockSpec((C, BN, CW), lambda i: (0, i, 0)))
        out_shape.append(jax.ShapeDtypeStruct((C, N, CW), jnp.float32))
    out_specs.append(pl.BlockSpec((G, DIM), lambda i: (0, 0)))
    out_shape.append(jax.ShapeDtypeStruct((G, DIM), jnp.float32))

    return pl.pallas_call(
        body,
        grid=(NB,),
        in_specs=[
            pl.BlockSpec((BN, DIM), lambda i: (i, 0)),
            pl.BlockSpec((2, DIM), lambda i: (0, 0)),
            pl.BlockSpec((1, DIM), lambda i: (0, 0)),
            pl.BlockSpec((1, DIM), lambda i: (0, 0)),
            pl.BlockSpec((1, 1, BN), lambda i: (i, 0, 0)),
        ],
        out_specs=out_specs,
        out_shape=out_shape,
        scratch_shapes=[pltpu.VMEM((G, DIM), jnp.float32)],
    )


@jax.jit
def kernel(x, edge_index, batch, eps, params):
    src = edge_index[0]
    dst = edge_index[1]
    pad = E_PAD - E
    dst_p = jnp.pad(dst, (0, pad))
    src_p = jnp.pad(src, (0, pad), constant_values=N)
    coff = (jnp.arange(4, dtype=jnp.int32) * N)[:, None]
    dstoff4 = dst_p[None, :] + coff
    dstoff2 = dstoff4[:2]
    batch3 = batch.reshape(NB, 1, BN)

    agg2 = _make_agg(2)
    agg4 = _make_agg(4)
    norm_mid = _make_norm(True)
    norm_last = _make_norm(False)

    h4 = x.reshape(N, 2, CW).transpose(1, 0, 2)
    xs, pools = [], []
    for i, p in enumerate(params):
        C = 2 if i == 0 else 4
        agg = agg2 if i == 0 else agg4
        dstoff = dstoff2 if i == 0 else dstoff4
        pooled = agg(h4.reshape(C * N, CW), dstoff, src_p).reshape(C, N, CW)
        scale = (1.0 + eps[i]).reshape(1, 1)
        hh, stats = _make_mlp(C)(
            scale, pooled, h4,
            p["W1"], p["b1"].reshape(1, DIM),
            p["W2"], p["b2"].reshape(1, DIM),
        )
        gam = p["gamma"].reshape(1, DIM)
        bet = p["beta"].reshape(1, DIM)
        if i < L - 1:
            h, h4, pool = norm_mid(hh, stats, gam, bet, batch3)
        else:
            h, pool = norm_last(hh, stats, gam, bet, batch3)
        xs.append(h)
        pools.append(pool)
    return (jnp.concatenate(pools, axis=1), jnp.concatenate(xs, axis=1))


# speedup vs baseline: 2.7039x; 1.0082x over previous
import functools

import jax
import jax.numpy as jnp
from jax import lax
from jax.experimental import pallas as pl
from jax.experimental.pallas import tpu as pltpu
from jax.experimental.pallas import tpu_sc as plsc

N = 10000
E = 160000
D_IN = 256
DIM = 512
L = 4
G = 64

NC = 2
NS = 16
CW = 128
EB = 128
SENT = 8

NBUF = 2
E_PAD = ((E + NS * EB * NBUF - 1) // (NS * EB * NBUF)) * (NS * EB * NBUF)
BN = 1000
NB = N // BN


def _make_agg(C):
    chunks_per_core = C // NC
    ebs = E_PAD // NS
    nb = ebs // EB
    rps = N // NS

    mesh = plsc.VectorSubcoreMesh(
        core_axis_name="c", subcore_axis_name="s", num_cores=NC, num_subcores=NS
    )

    @functools.partial(
        pl.kernel,
        out_type=jax.ShapeDtypeStruct((C * N, CW), jnp.float32),
        mesh=mesh,
        compiler_params=pltpu.CompilerParams(use_tc_tiling_on_sc=False),
        scratch_types=[
            [pltpu.VMEM((NBUF, EB), jnp.int32) for _ in range(2)],
            [pltpu.VMEM((EB, CW), jnp.float32) for _ in range(NBUF)],
            pltpu.VMEM_SHARED((N + SENT, CW), jnp.float32),
            [pltpu.SemaphoreType.DMA for _ in range(NBUF)],
            [pltpu.SemaphoreType.DMA for _ in range(NBUF)],
            pltpu.SemaphoreType.DMA,
        ],
    )
    def agg(h_hbm, dstoff_hbm, src_hbm, out_hbm,
            idx, rows, acc_sh, gsem, isem, ssem):
        cid = lax.axis_index("c")
        sid = lax.axis_index("s")
        dstb, srcb = idx

        def istart(chunk, b, j):
            pltpu.async_copy(dstoff_hbm.at[chunk, sid, b], dstb.at[j], isem[j])
            pltpu.async_copy(src_hbm.at[sid, b], srcb.at[j], isem[j])

        def iwait(chunk, b, j):
            pltpu.make_async_copy(
                dstoff_hbm.at[chunk, sid, b], dstb.at[j], isem[j]).wait()
            pltpu.make_async_copy(src_hbm.at[sid, b], srcb.at[j], isem[j]).wait()

        def gstart(j):
            pltpu.async_copy(h_hbm.at[dstb.at[j]], rows[j], gsem[j])

        def gwait(j):
            pltpu.make_async_copy(h_hbm.at[dstb.at[j]], rows[j], gsem[j]).wait()

        for k in range(chunks_per_core):
            chunk = cid * chunks_per_core + k

            def zrow(r, _):
                for j in range(CW // 16):
                    rows[0][r, pl.ds(j * 16, 16)] = jnp.zeros((16,), jnp.float32)
                return 0
            lax.fori_loop(0, EB, zrow, 0)
            nfull = rps // EB
            for t in range(nfull):
                pltpu.sync_copy(rows[0], acc_sh.at[pl.ds(sid * rps + t * EB, EB)])
            rem = rps - nfull * EB
            if rem:
                pltpu.sync_copy(
                    rows[0].at[pl.ds(0, rem)],
                    acc_sh.at[pl.ds(sid * rps + nfull * EB, rem)],
                )
            plsc.subcore_barrier()

            for j in range(NBUF):
                istart(chunk, j, j)
            for j in range(NBUF):
                iwait(chunk, j, j)
                gstart(j)

            def outer(g, _):
                for j in range(NBUF):
                    b = g * NBUF + j
                    gwait(j)
                    sc = pltpu.async_copy(
                        rows[j], acc_sh.at[srcb.at[j]], ssem, add=True)
                    sc.wait()

                    @pl.when(b + NBUF < nb)
                    def _():
                        istart(chunk, b + NBUF, j)
                        iwait(chunk, b + NBUF, j)
                        gstart(j)
                return 0
            lax.fori_loop(0, nb // NBUF, outer, 0)
            plsc.subcore_barrier()

            pltpu.sync_copy(
                acc_sh.at[pl.ds(sid * rps, rps)],
                out_hbm.at[pl.ds(chunk * N + sid * rps, rps)],
            )
            if k + 1 < chunks_per_core:
                plsc.subcore_barrier()

    return agg


def _make_mlp(C):
    din = C * CW

    def body(scale_ref, pooled_ref, h_ref, w1_ref, b1_ref, w2_ref, b2_ref,
             hh_ref, stats_ref, acc_ref):
        i = pl.program_id(0)
        s = scale_ref[0, 0]
        p = jnp.concatenate([pooled_ref[c] for c in range(C)], axis=-1)
        hcat = jnp.concatenate([h_ref[c] for c in range(C)], axis=-1)
        a = p + s * hcat
        z = jnp.dot(a, w1_ref[...], preferred_element_type=jnp.float32)
        z = jnp.maximum(z + b1_ref[0][None, :], 0.0)
        hh = jnp.dot(z, w2_ref[...], preferred_element_type=jnp.float32)
        hh = jnp.maximum(hh + b2_ref[0][None, :], 0.0)
        hh_ref[...] = hh

        @pl.when(i == 0)
        def _():
            acc_ref[...] = jnp.zeros_like(acc_ref)
        acc_ref[0, :] += jnp.sum(hh, axis=0)
        acc_ref[1, :] += jnp.sum(hh * hh, axis=0)

        @pl.when(i == NB - 1)
        def _():
            stats_ref[...] = acc_ref[...]

    return pl.pallas_call(
        body,
        grid=(NB,),
        in_specs=[
            pl.BlockSpec(memory_space=pltpu.SMEM),
            pl.BlockSpec((C, BN, CW), lambda i: (0, i, 0)),
            pl.BlockSpec((C, BN, CW), lambda i: (0, i, 0)),
            pl.BlockSpec((C * CW, DIM), lambda i: (0, 0)),
            pl.BlockSpec((1, DIM), lambda i: (0, 0)),
            pl.BlockSpec((DIM, DIM), lambda i: (0, 0)),
            pl.BlockSpec((1, DIM), lambda i: (0, 0)),
        ],
        out_specs=[
            pl.BlockSpec((BN, DIM), lambda i: (i, 0)),
            pl.BlockSpec((2, DIM), lambda i: (0, 0)),
        ],
        out_shape=[
            jax.ShapeDtypeStruct((N, DIM), jnp.float32),
            jax.ShapeDtypeStruct((2, DIM), jnp.float32),
        ],
        scratch_shapes=[pltpu.VMEM((2, DIM), jnp.float32)],
    )


def _make_norm(with_relayout):
    C = DIM // CW

    def body(hh_ref, stats_ref, gamma_ref, beta_ref, batch_ref, *refs):
        if with_relayout:
            xs_ref, h4_ref, pool_ref, acc_ref = refs
        else:
            xs_ref, pool_ref, acc_ref = refs
        i = pl.program_id(0)
        mean = stats_ref[0, :] * (1.0 / N)
        var = stats_ref[1, :] * (1.0 / N) - mean * mean
        g = gamma_ref[0, :] * lax.rsqrt(var + 1e-5)
        b = beta_ref[0, :] - mean * g
        hn = hh_ref[...] * g[None, :] + b[None, :]
        xs_ref[...] = hn
        if with_relayout:
            for c in range(C):
                h4_ref[c] = hn[:, c * CW:(c + 1) * CW]
        bid = batch_ref[0, 0, :]
        onehot = (bid[:, None] == lax.broadcasted_iota(jnp.int32, (1, G), 1)
                  ).astype(jnp.float32)

        @pl.when(i == 0)
        def _():
            acc_ref[...] = jnp.zeros_like(acc_ref)
        acc_ref[...] += lax.dot_general(
            onehot, hn, (((0,), (0,)), ((), ())),
            preferred_element_type=jnp.float32)

        @pl.when(i == NB - 1)
        def _():
            pool_ref[...] = acc_ref[...]

    out_specs = [pl.BlockSpec((BN, DIM), lambda i: (i, 0))]
    out_shape = [jax.ShapeDtypeStruct((N, DIM), jnp.float32)]
    if with_relayout:
        out_specs.append(pl.BlockSpec((C, BN, CW), lambda i: (0, i, 0)))
        out_shape.append(jax.ShapeDtypeStruct((C, N, CW), jnp.float32))
    out_specs.append(pl.BlockSpec((G, DIM), lambda i: (0, 0)))
    out_shape.append(jax.ShapeDtypeStruct((G, DIM), jnp.float32))

    return pl.pallas_call(
        body,
        grid=(NB,),
        in_specs=[
            pl.BlockSpec((BN, DIM), lambda i: (i, 0)),
            pl.BlockSpec((2, DIM), lambda i: (0, 0)),
            pl.BlockSpec((1, DIM), lambda i: (0, 0)),
            pl.BlockSpec((1, DIM), lambda i: (0, 0)),
            pl.BlockSpec((1, 1, BN), lambda i: (i, 0, 0)),
        ],
        out_specs=out_specs,
        out_shape=out_shape,
        scratch_shapes=[pltpu.VMEM((G, DIM), jnp.float32)],
    )


@jax.jit
def kernel(x, edge_index, batch, eps, params):
    src = edge_index[0]
    dst = edge_index[1]
    pad = E_PAD - E
    nb = E_PAD // (NS * EB)
    dst_p = jnp.pad(dst, (0, pad))
    src_p = jnp.pad(src, (0, pad), constant_values=N).reshape(NS, nb, EB)
    coff = (jnp.arange(4, dtype=jnp.int32) * N)[:, None]
    dstoff4 = (dst_p[None, :] + coff).reshape(4, NS, nb, EB)
    dstoff2 = dstoff4[:2]
    batch3 = batch.reshape(NB, 1, BN)

    agg2 = _make_agg(2)
    agg4 = _make_agg(4)
    norm_mid = _make_norm(True)
    norm_last = _make_norm(False)

    h4 = x.reshape(N, 2, CW).transpose(1, 0, 2)
    xs, pools = [], []
    for i, p in enumerate(params):
        C = 2 if i == 0 else 4
        agg = agg2 if i == 0 else agg4
        dstoff = dstoff2 if i == 0 else dstoff4
        pooled = agg(h4.reshape(C * N, CW), dstoff, src_p).reshape(C, N, CW)
        scale = (1.0 + eps[i]).reshape(1, 1)
        hh, stats = _make_mlp(C)(
            scale, pooled, h4,
            p["W1"], p["b1"].reshape(1, DIM),
            p["W2"], p["b2"].reshape(1, DIM),
        )
        gam = p["gamma"].reshape(1, DIM)
        bet = p["beta"].reshape(1, DIM)
        if i < L - 1:
            h, h4, pool = norm_mid(hh, stats, gam, bet, batch3)
        else:
            h, pool = norm_last(hh, stats, gam, bet, batch3)
        xs.append(h)
        pools.append(pool)
    return (jnp.concatenate(pools, axis=1), jnp.concatenate(xs, axis=1))


# speedup vs baseline: 2.9018x; 1.0732x over previous
import functools

import jax
import jax.numpy as jnp
from jax import lax
from jax.experimental import pallas as pl
from jax.experimental.pallas import tpu as pltpu
from jax.experimental.pallas import tpu_sc as plsc

N = 10000
E = 160000
D_IN = 256
DIM = 512
L = 4
G = 64

NC = 2
NS = 16
CW = 128
EB = 64
SENT = 8

NBUF = 3
E_PAD = ((E + NS * EB * NBUF - 1) // (NS * EB * NBUF)) * (NS * EB * NBUF)
BN = 1000
NB = N // BN


def _make_agg(C):
    chunks_per_core = C // NC
    ebs = E_PAD // NS
    nb = ebs // EB
    rps = N // NS

    mesh = plsc.VectorSubcoreMesh(
        core_axis_name="c", subcore_axis_name="s", num_cores=NC, num_subcores=NS
    )

    @functools.partial(
        pl.kernel,
        out_type=jax.ShapeDtypeStruct((C * N, CW), jnp.float32),
        mesh=mesh,
        compiler_params=pltpu.CompilerParams(use_tc_tiling_on_sc=False),
        scratch_types=[
            [pltpu.VMEM((NBUF, EB), jnp.int32) for _ in range(2)],
            [pltpu.VMEM((EB, CW), jnp.float32) for _ in range(NBUF)],
            pltpu.VMEM_SHARED((N + SENT, CW), jnp.float32),
            [pltpu.SemaphoreType.DMA for _ in range(NBUF)],
            [pltpu.SemaphoreType.DMA for _ in range(NBUF)],
            [pltpu.SemaphoreType.DMA for _ in range(NBUF)],
        ],
    )
    def agg(h_hbm, dstoff_hbm, src_hbm, out_hbm,
            idx, rows, acc_sh, gsem, isem, ssem):
        cid = lax.axis_index("c")
        sid = lax.axis_index("s")
        dstb, srcb = idx

        def istart(chunk, b, j):
            pltpu.async_copy(dstoff_hbm.at[chunk, sid, b], dstb.at[j], isem[j])
            pltpu.async_copy(src_hbm.at[sid, b], srcb.at[j], isem[j])

        def iwait(chunk, b, j):
            pltpu.make_async_copy(
                dstoff_hbm.at[chunk, sid, b], dstb.at[j], isem[j]).wait()
            pltpu.make_async_copy(src_hbm.at[sid, b], srcb.at[j], isem[j]).wait()

        def gstart(j):
            pltpu.async_copy(h_hbm.at[dstb.at[j]], rows[j], gsem[j])

        def gwait(j):
            pltpu.make_async_copy(h_hbm.at[dstb.at[j]], rows[j], gsem[j]).wait()

        def scstart(j):
            pltpu.async_copy(rows[j], acc_sh.at[srcb.at[j]], ssem[j], add=True)

        def scwait(j):
            pltpu.make_async_copy(rows[j], acc_sh.at[srcb.at[j]], ssem[j]).wait()

        for k in range(chunks_per_core):
            chunk = cid * chunks_per_core + k

            def zrow(r, _):
                for j in range(CW // 16):
                    rows[0][r, pl.ds(j * 16, 16)] = jnp.zeros((16,), jnp.float32)
                return 0
            lax.fori_loop(0, EB, zrow, 0)
            nfull = rps // EB
            for t in range(nfull):
                pltpu.sync_copy(rows[0], acc_sh.at[pl.ds(sid * rps + t * EB, EB)])
            rem = rps - nfull * EB
            if rem:
                pltpu.sync_copy(
                    rows[0].at[pl.ds(0, rem)],
                    acc_sh.at[pl.ds(sid * rps + nfull * EB, rem)],
                )
            plsc.subcore_barrier()

            for j in range(NBUF):
                istart(chunk, j, j)
            for j in range(NBUF):
                iwait(chunk, j, j)
                gstart(j)

            def outer(g, _):
                for j in range(NBUF):
                    b = g * NBUF + j
                    gwait(j)
                    scstart(j)
                    scwait(j)

                    @pl.when(b + NBUF < nb)
                    def _():
                        istart(chunk, b + NBUF, j)
                        iwait(chunk, b + NBUF, j)
                        gstart(j)
                return 0
            lax.fori_loop(0, nb // NBUF, outer, 0)
            plsc.subcore_barrier()

            pltpu.sync_copy(
                acc_sh.at[pl.ds(sid * rps, rps)],
                out_hbm.at[pl.ds(chunk * N + sid * rps, rps)],
            )
            if k + 1 < chunks_per_core:
                plsc.subcore_barrier()

    return agg


def _make_mlp(C):
    din = C * CW

    def body(scale_ref, pooled_ref, h_ref, w1_ref, b1_ref, w2_ref, b2_ref,
             hh_ref, stats_ref, acc_ref):
        i = pl.program_id(0)
        s = scale_ref[0, 0]
        p = jnp.concatenate([pooled_ref[c] for c in range(C)], axis=-1)
        hcat = jnp.concatenate([h_ref[c] for c in range(C)], axis=-1)
        a = p + s * hcat
        z = jnp.dot(a, w1_ref[...], preferred_element_type=jnp.float32)
        z = jnp.maximum(z + b1_ref[0][None, :], 0.0)
        hh = jnp.dot(z, w2_ref[...], preferred_element_type=jnp.float32)
        hh = jnp.maximum(hh + b2_ref[0][None, :], 0.0)
        hh_ref[...] = hh

        @pl.when(i == 0)
        def _():
            acc_ref[...] = jnp.zeros_like(acc_ref)
        acc_ref[0, :] += jnp.sum(hh, axis=0)
        acc_ref[1, :] += jnp.sum(hh * hh, axis=0)

        @pl.when(i == NB - 1)
        def _():
            stats_ref[...] = acc_ref[...]

    return pl.pallas_call(
        body,
        grid=(NB,),
        in_specs=[
            pl.BlockSpec(memory_space=pltpu.SMEM),
            pl.BlockSpec((C, BN, CW), lambda i: (0, i, 0)),
            pl.BlockSpec((C, BN, CW), lambda i: (0, i, 0)),
            pl.BlockSpec((C * CW, DIM), lambda i: (0, 0)),
            pl.BlockSpec((1, DIM), lambda i: (0, 0)),
            pl.BlockSpec((DIM, DIM), lambda i: (0, 0)),
            pl.BlockSpec((1, DIM), lambda i: (0, 0)),
        ],
        out_specs=[
            pl.BlockSpec((BN, DIM), lambda i: (i, 0)),
            pl.BlockSpec((2, DIM), lambda i: (0, 0)),
        ],
        out_shape=[
            jax.ShapeDtypeStruct((N, DIM), jnp.float32),
            jax.ShapeDtypeStruct((2, DIM), jnp.float32),
        ],
        scratch_shapes=[pltpu.VMEM((2, DIM), jnp.float32)],
    )


def _make_norm(with_relayout):
    C = DIM // CW

    def body(hh_ref, stats_ref, gamma_ref, beta_ref, batch_ref, *refs):
        if with_relayout:
            xs_ref, h4_ref, pool_ref, acc_ref = refs
        else:
            xs_ref, pool_ref, acc_ref = refs
        i = pl.program_id(0)
        mean = stats_ref[0, :] * (1.0 / N)
        var = stats_ref[1, :] * (1.0 / N) - mean * mean
        g = gamma_ref[0, :] * lax.rsqrt(var + 1e-5)
        b = beta_ref[0, :] - mean * g
        hn = hh_ref[...] * g[None, :] + b[None, :]
        xs_ref[...] = hn
        if with_relayout:
            for c in range(C):
                h4_ref[c] = hn[:, c * CW:(c + 1) * CW]
        bid = batch_ref[0, 0, :]
        onehot = (bid[:, None] == lax.broadcasted_iota(jnp.int32, (1, G), 1)
                  ).astype(jnp.float32)

        @pl.when(i == 0)
        def _():
            acc_ref[...] = jnp.zeros_like(acc_ref)
        acc_ref[...] += lax.dot_general(
            onehot, hn, (((0,), (0,)), ((), ())),
            preferred_element_type=jnp.float32)

        @pl.when(i == NB - 1)
        def _():
            pool_ref[...] = acc_ref[...]

    out_specs = [pl.BlockSpec((BN, DIM), lambda i: (i, 0))]
    out_shape = [jax.ShapeDtypeStruct((N, DIM), jnp.float32)]
    if with_relayout:
        out_specs.append(pl.BlockSpec((C, BN, CW), lambda i: (0, i, 0)))
        out_shape.append(jax.ShapeDtypeStruct((C, N, CW), jnp.float32))
    out_specs.append(pl.BlockSpec((G, DIM), lambda i: (0, 0)))
    out_shape.append(jax.ShapeDtypeStruct((G, DIM), jnp.float32))

    return pl.pallas_call(
        body,
        grid=(NB,),
        in_specs=[
            pl.BlockSpec((BN, DIM), lambda i: (i, 0)),
            pl.BlockSpec((2, DIM), lambda i: (0, 0)),
            pl.BlockSpec((1, DIM), lambda i: (0, 0)),
            pl.BlockSpec((1, DIM), lambda i: (0, 0)),
            pl.BlockSpec((1, 1, BN), lambda i: (i, 0, 0)),
        ],
        out_specs=out_specs,
        out_shape=out_shape,
        scratch_shapes=[pltpu.VMEM((G, DIM), jnp.float32)],
    )


@jax.jit
def kernel(x, edge_index, batch, eps, params):
    src = edge_index[0]
    dst = edge_index[1]
    pad = E_PAD - E
    nb = E_PAD // (NS * EB)
    dst_p = jnp.pad(dst, (0, pad))
    src_p = jnp.pad(src, (0, pad), constant_values=N).reshape(NS, nb, EB)
    coff = (jnp.arange(4, dtype=jnp.int32) * N)[:, None]
    dstoff4 = (dst_p[None, :] + coff).reshape(4, NS, nb, EB)
    dstoff2 = dstoff4[:2]
    batch3 = batch.reshape(NB, 1, BN)

    agg2 = _make_agg(2)
    agg4 = _make_agg(4)
    norm_mid = _make_norm(True)
    norm_last = _make_norm(False)

    h4 = x.reshape(N, 2, CW).transpose(1, 0, 2)
    xs, pools = [], []
    for i, p in enumerate(params):
        C = 2 if i == 0 else 4
        agg = agg2 if i == 0 else agg4
        dstoff = dstoff2 if i == 0 else dstoff4
        pooled = agg(h4.reshape(C * N, CW), dstoff, src_p).reshape(C, N, CW)
        scale = (1.0 + eps[i]).reshape(1, 1)
        hh, stats = _make_mlp(C)(
            scale, pooled, h4,
            p["W1"], p["b1"].reshape(1, DIM),
            p["W2"], p["b2"].reshape(1, DIM),
        )
        gam = p["gamma"].reshape(1, DIM)
        bet = p["beta"].reshape(1, DIM)
        if i < L - 1:
            h, h4, pool = norm_mid(hh, stats, gam, bet, batch3)
        else:
            h, pool = norm_last(hh, stats, gam, bet, batch3)
        xs.append(h)
        pools.append(pool)
    return (jnp.concatenate(pools, axis=1), jnp.concatenate(xs, axis=1))
